# baseline (speedup 1.0000x reference)
import functools

import jax
import jax.numpy as jnp
from jax import lax
from jax.experimental import pallas as pl
from jax.experimental.pallas import tpu as pltpu
from jax.experimental.pallas import tpu_sc as plsc

N_TEX = 16
N_FEAT = 16
TEX_DIM = 512
N_LAYERS = 4
H = 512
W = 512
P = H * W
NW = 32
WPL = NW // N_LAYERS
PPW = P // WPL
B = 1024
NCHUNK = PPW // B
NROW = (4 * B) // 128
RPT = B // 128


TPB = 128


def _tr_body(data4, table1, in_v, out_v, sem):
    c = lax.axis_index("c")
    s = lax.axis_index("s")
    wid = s * 2 + c
    lane = lax.broadcasted_iota(jnp.int32, (16,), 0)
    colf = [jnp.full((16,), f, jnp.int32) for f in range(N_FEAT)]

    def blk(bi, carry):
        g = wid * TPB + bi
        t = g // 256
        rem = g % 256
        yt = rem // 4
        xt = rem % 4
        pltpu.async_copy(
            data4.at[t, :, pl.ds(yt * 8, 8), pl.ds(xt * 128, 128)],
            in_v, sem).wait()

        def ys_body(ys, carry2):
            def xc_body(xc, carry3):
                row = ys * 128 + xc * 16
                for f in range(N_FEAT):
                    v = in_v[f, ys, pl.ds(xc * 16, 16)]
                    plsc.store_scatter(out_v, [lane + row, colf[f]], v)
                return carry3
            lax.fori_loop(0, 8, xc_body, 0)
            return carry2
        lax.fori_loop(0, 8, ys_body, 0)

        rb = (t * TEX_DIM + yt * 8) * TEX_DIM + xt * 128
        for ys in range(8):
            pltpu.sync_copy(
                out_v.at[pl.ds(ys * 128, 128)],
                table1.at[pl.ds(rb + ys * TEX_DIM, 128)])
        return carry

    lax.fori_loop(0, TPB, blk, 0)


def _sc_body(table, uvf, maskf, out, uv_v, tid_v, idx_v, w_v, rows_v, outT_v, sem):
    c = lax.axis_index("c")
    s = lax.axis_index("s")
    wid = s * 2 + c
    layer = wid // WPL
    base0 = (wid % WPL) * PPW
    lane = lax.broadcasted_iota(jnp.int32, (16,), 0)
    laneB = lane * B

    def chunk_body(ci, carry):
        base = base0 + ci * B
        pltpu.sync_copy(uvf.at[pl.ds(2 * layer, 2), pl.ds(base, B)], uv_v)
        pltpu.sync_copy(maskf.at[layer, pl.ds(base, B)], tid_v)

        def vec_body(r, carry2):
            for k in range(8):
                o = r * 128 + k * 16
                x = uv_v[0, pl.ds(o, 16)]
                y = uv_v[1, pl.ds(o, 16)]
                t = tid_v[pl.ds(o, 16)]
                ix = ((x + 1.0) * float(TEX_DIM) - 1.0) * 0.5
                iy = ((y + 1.0) * float(TEX_DIM) - 1.0) * 0.5
                ixf = ix.astype(jnp.int32).astype(jnp.float32)
                ix0 = jnp.where(ixf > ix, ixf - 1.0, ixf)
                iyf = iy.astype(jnp.int32).astype(jnp.float32)
                iy0 = jnp.where(iyf > iy, iyf - 1.0, iyf)
                wx1 = ix - ix0
                wy1 = iy - iy0
                wx0 = 1.0 - wx1
                wy0 = 1.0 - wy1
                valid = jnp.where(t >= 1, 1.0, 0.0)
                wy0 = wy0 * valid
                wy1 = wy1 * valid
                w_v[0, pl.ds(o, 16)] = wy0 * wx0
                w_v[1, pl.ds(o, 16)] = wy0 * wx1
                w_v[2, pl.ds(o, 16)] = wy1 * wx0
                w_v[3, pl.ds(o, 16)] = wy1 * wx1
                hi = float(TEX_DIM - 1)
                ix0c = jnp.clip(ix0, 0.0, hi).astype(jnp.int32)
                ix1c = jnp.clip(ix0 + 1.0, 0.0, hi).astype(jnp.int32)
                iy0c = jnp.clip(iy0, 0.0, hi).astype(jnp.int32)
                iy1c = jnp.clip(iy0 + 1.0, 0.0, hi).astype(jnp.int32)
                tb = t << 18
                r0 = tb + (iy0c << 9)
                r1 = tb + (iy1c << 9)
                col = pl.ds(k * 16, 16)
                idx_v[0 * RPT + r, col] = r0 + ix0c
                idx_v[1 * RPT + r, col] = r0 + ix1c
                idx_v[2 * RPT + r, col] = r1 + ix0c
                idx_v[3 * RPT + r, col] = r1 + ix1c
            return carry2

        lax.fori_loop(0, RPT, vec_body, 0)

        handles = []
        for j in range(NROW):
            handles.append(
                pltpu.async_copy(
                    table.at[idx_v.at[j]],
                    rows_v.at[pl.ds(j * 128, 128)],
                    sem,
                )
            )
        for h in handles:
            h.wait()

        def pix_body(g, carry2):
            o = g * 16
            w0v = w_v[0, pl.ds(o, 16)]
            w1v = w_v[1, pl.ds(o, 16)]
            w2v = w_v[2, pl.ds(o, 16)]
            w3v = w_v[3, pl.ds(o, 16)]
            for j in range(16):
                p = o + j
                acc = (rows_v[p, :] * w0v[j] + rows_v[B + p, :] * w1v[j]
                       + rows_v[2 * B + p, :] * w2v[j]
                       + rows_v[3 * B + p, :] * w3v[j])
                plsc.store_scatter(outT_v, [laneB + p], acc)
            return carry2

        lax.fori_loop(0, B // 16, pix_body, 0)

        for f in range(N_FEAT):
            pltpu.sync_copy(outT_v.at[pl.ds(f * B, B)],
                            out.at[layer, f, pl.ds(base, B)])
        return carry

    lax.fori_loop(0, NCHUNK, chunk_body, 0)


@functools.partial(jax.jit, static_argnames=())
def _run(data, uvf, maskf):
    mesh = plsc.VectorSubcoreMesh(
        core_axis_name="c", subcore_axis_name="s",
        num_cores=2, num_subcores=16)
    tr = pl.kernel(
        _tr_body,
        out_type=jax.ShapeDtypeStruct((N_TEX * TEX_DIM * TEX_DIM, N_FEAT),
                                      jnp.float32),
        mesh=mesh,
        scratch_types=[
            pltpu.VMEM((N_FEAT, 8, 128), jnp.float32),
            pltpu.VMEM((8 * 128, N_FEAT), jnp.float32),
            pltpu.SemaphoreType.DMA,
        ],
        compiler_params=pltpu.CompilerParams(
            needs_layout_passes=False, use_tc_tiling_on_sc=False),
    )
    table = tr(data)
    fn = pl.kernel(
        _sc_body,
        out_type=jax.ShapeDtypeStruct((N_LAYERS, N_FEAT, P), jnp.float32),
        mesh=mesh,
        scratch_types=[
            pltpu.VMEM((2, B), jnp.float32),
            pltpu.VMEM((B,), jnp.int32),
            pltpu.VMEM((NROW, 128), jnp.int32),
            pltpu.VMEM((4, B), jnp.float32),
            pltpu.VMEM((4 * B, N_FEAT), jnp.float32),
            pltpu.VMEM((N_FEAT * B,), jnp.float32),
            pltpu.SemaphoreType.DMA,
        ],
        compiler_params=pltpu.CompilerParams(
            needs_layout_passes=False, use_tc_tiling_on_sc=False),
    )
    return fn(table, uvf, maskf)


def kernel(data, uv_inputs, mask_inputs, world_positions, extrinsics):
    uvf = uv_inputs.reshape(2 * N_LAYERS, P)
    maskf = mask_inputs.reshape(N_LAYERS, P).astype(jnp.int32)
    res = _run(data, uvf, maskf)
    return res.reshape(1, N_LAYERS * N_FEAT, H, W)


# speedup vs baseline: 1.1125x; 1.1125x over previous
import functools

import jax
import jax.numpy as jnp
from jax import lax
from jax.experimental import pallas as pl
from jax.experimental.pallas import tpu as pltpu
from jax.experimental.pallas import tpu_sc as plsc

N_TEX = 16
N_FEAT = 16
TEX_DIM = 512
N_LAYERS = 4
H = 512
W = 512
P = H * W
NW = 32
WPL = NW // N_LAYERS
PPW = P // WPL
B = 1024
NCHUNK = PPW // B
NROW = (4 * B) // 128
RPT = B // 128


TPB = 128


def _tr_body(data4, table1, in_v, out_v, sem):
    c = lax.axis_index("c")
    s = lax.axis_index("s")
    wid = s * 2 + c
    lane = lax.broadcasted_iota(jnp.int32, (16,), 0)
    colf = [jnp.full((16,), f, jnp.int32) for f in range(N_FEAT)]

    def blk(bi, carry):
        g = wid * TPB + bi
        t = g // 256
        rem = g % 256
        yt = rem // 4
        xt = rem % 4
        pltpu.async_copy(
            data4.at[t, :, yt, xt], in_v, sem).wait()

        def ys_body(ys, carry2):
            def xc_body(xc, carry3):
                row = ys * 128 + xc * 16
                for f in range(N_FEAT):
                    v = in_v[f, ys, pl.ds(xc * 16, 16)]
                    plsc.store_scatter(out_v, [lane + row, colf[f]], v)
                return carry3
            lax.fori_loop(0, 8, xc_body, 0)
            return carry2
        lax.fori_loop(0, 8, ys_body, 0)

        rb = (t * TEX_DIM + yt * 8) * TEX_DIM + xt * 128
        for ys in range(8):
            pltpu.sync_copy(
                out_v.at[pl.ds(ys * 128, 128)],
                table1.at[pl.ds(rb + ys * TEX_DIM, 128)])
        return carry

    lax.fori_loop(0, TPB, blk, 0)


def _sc_body(table, uvf, maskf, out, uv_v, tid_v, idx_v, w_v, rows_v, outT_v, sem):
    c = lax.axis_index("c")
    s = lax.axis_index("s")
    wid = s * 2 + c
    layer = wid // WPL
    base0 = (wid % WPL) * PPW
    lane = lax.broadcasted_iota(jnp.int32, (16,), 0)
    laneB = lane * B

    def chunk_body(ci, carry):
        base = base0 + ci * B
        pltpu.sync_copy(uvf.at[pl.ds(2 * layer, 2), pl.ds(base, B)], uv_v)
        pltpu.sync_copy(maskf.at[layer, pl.ds(base, B)], tid_v)

        def vec_body(r, carry2):
            for k in range(8):
                o = r * 128 + k * 16
                x = uv_v[0, pl.ds(o, 16)]
                y = uv_v[1, pl.ds(o, 16)]
                t = tid_v[pl.ds(o, 16)]
                ix = ((x + 1.0) * float(TEX_DIM) - 1.0) * 0.5
                iy = ((y + 1.0) * float(TEX_DIM) - 1.0) * 0.5
                ixf = ix.astype(jnp.int32).astype(jnp.float32)
                ix0 = jnp.where(ixf > ix, ixf - 1.0, ixf)
                iyf = iy.astype(jnp.int32).astype(jnp.float32)
                iy0 = jnp.where(iyf > iy, iyf - 1.0, iyf)
                wx1 = ix - ix0
                wy1 = iy - iy0
                wx0 = 1.0 - wx1
                wy0 = 1.0 - wy1
                valid = jnp.where(t >= 1, 1.0, 0.0)
                wy0 = wy0 * valid
                wy1 = wy1 * valid
                w_v[0, pl.ds(o, 16)] = wy0 * wx0
                w_v[1, pl.ds(o, 16)] = wy0 * wx1
                w_v[2, pl.ds(o, 16)] = wy1 * wx0
                w_v[3, pl.ds(o, 16)] = wy1 * wx1
                hi = float(TEX_DIM - 1)
                ix0c = jnp.clip(ix0, 0.0, hi).astype(jnp.int32)
                ix1c = jnp.clip(ix0 + 1.0, 0.0, hi).astype(jnp.int32)
                iy0c = jnp.clip(iy0, 0.0, hi).astype(jnp.int32)
                iy1c = jnp.clip(iy0 + 1.0, 0.0, hi).astype(jnp.int32)
                tb = t << 18
                r0 = tb + (iy0c << 9)
                r1 = tb + (iy1c << 9)
                col = pl.ds(k * 16, 16)
                idx_v[0 * RPT + r, col] = r0 + ix0c
                idx_v[1 * RPT + r, col] = r0 + ix1c
                idx_v[2 * RPT + r, col] = r1 + ix0c
                idx_v[3 * RPT + r, col] = r1 + ix1c
            return carry2

        lax.fori_loop(0, RPT, vec_body, 0)

        handles = []
        for j in range(NROW):
            handles.append(
                pltpu.async_copy(
                    table.at[idx_v.at[j]],
                    rows_v.at[pl.ds(j * 128, 128)],
                    sem,
                )
            )
        for h in handles:
            h.wait()

        def pix_body(g, carry2):
            o = g * 16
            w0v = w_v[0, pl.ds(o, 16)]
            w1v = w_v[1, pl.ds(o, 16)]
            w2v = w_v[2, pl.ds(o, 16)]
            w3v = w_v[3, pl.ds(o, 16)]
            for j in range(16):
                p = o + j
                acc = (rows_v[p, :] * w0v[j] + rows_v[B + p, :] * w1v[j]
                       + rows_v[2 * B + p, :] * w2v[j]
                       + rows_v[3 * B + p, :] * w3v[j])
                plsc.store_scatter(outT_v, [laneB + p], acc)
            return carry2

        lax.fori_loop(0, B // 16, pix_body, 0)

        for f in range(N_FEAT):
            pltpu.sync_copy(outT_v.at[pl.ds(f * B, B)],
                            out.at[layer, f, pl.ds(base, B)])
        return carry

    lax.fori_loop(0, NCHUNK, chunk_body, 0)


@functools.partial(jax.jit, static_argnames=())
def _run(data, uvf, maskf):
    mesh = plsc.VectorSubcoreMesh(
        core_axis_name="c", subcore_axis_name="s",
        num_cores=2, num_subcores=16)
    tr = pl.kernel(
        _tr_body,
        out_type=jax.ShapeDtypeStruct((N_TEX * TEX_DIM * TEX_DIM, N_FEAT),
                                      jnp.float32),
        mesh=mesh,
        scratch_types=[
            pltpu.VMEM((N_FEAT, 8, 128), jnp.float32),
            pltpu.VMEM((8 * 128, N_FEAT), jnp.float32),
            pltpu.SemaphoreType.DMA,
        ],
        compiler_params=pltpu.CompilerParams(
            needs_layout_passes=False, use_tc_tiling_on_sc=False),
    )
    table = tr(data)
    fn = pl.kernel(
        _sc_body,
        out_type=jax.ShapeDtypeStruct((N_LAYERS, N_FEAT, P), jnp.float32),
        mesh=mesh,
        scratch_types=[
            pltpu.VMEM((2, B), jnp.float32),
            pltpu.VMEM((B,), jnp.int32),
            pltpu.VMEM((NROW, 128), jnp.int32),
            pltpu.VMEM((4, B), jnp.float32),
            pltpu.VMEM((4 * B, N_FEAT), jnp.float32),
            pltpu.VMEM((N_FEAT * B,), jnp.float32),
            pltpu.SemaphoreType.DMA,
        ],
        compiler_params=pltpu.CompilerParams(
            needs_layout_passes=False, use_tc_tiling_on_sc=False),
    )
    return fn(table, uvf, maskf)


def kernel(data, uv_inputs, mask_inputs, world_positions, extrinsics):
    uvf = uv_inputs.reshape(2 * N_LAYERS, P)
    maskf = mask_inputs.reshape(N_LAYERS, P).astype(jnp.int32)
    dataT = data.reshape(N_TEX, N_FEAT, TEX_DIM // 8, 8,
                         TEX_DIM // 128, 128).swapaxes(3, 4)
    res = _run(dataT, uvf, maskf)
    return res.reshape(1, N_LAYERS * N_FEAT, H, W)


# speedup vs baseline: 1.1149x; 1.0022x over previous
import functools

import jax
import jax.numpy as jnp
from jax import lax
from jax.experimental import pallas as pl
from jax.experimental.pallas import tpu as pltpu
from jax.experimental.pallas import tpu_sc as plsc

N_TEX = 16
N_FEAT = 16
TEX_DIM = 512
N_LAYERS = 4
H = 512
W = 512
P = H * W
NW = 32
WPL = NW // N_LAYERS
PPW = P // WPL
B = 1024
NCHUNK = PPW // B
NROW = (4 * B) // 128
RPT = B // 128


TPB = 128


def _tr_body(data4, table1, in_v, out_v, sem):
    c = lax.axis_index("c")
    s = lax.axis_index("s")
    wid = s * 2 + c
    lane = lax.broadcasted_iota(jnp.int32, (16,), 0)
    colf = [jnp.full((16,), f, jnp.int32) for f in range(N_FEAT)]

    def blk(bi, carry):
        g = wid * TPB + bi
        t = g // 256
        rem = g % 256
        yt = rem // 4
        xt = rem % 4
        pltpu.async_copy(
            data4.at[t, :, yt, xt], in_v, sem).wait()

        def ys_body(ys, carry2):
            def xc_body(xc, carry3):
                row = ys * 128 + xc * 16
                dst = out_v.at[pl.ds(row, 16), :]
                for f in range(N_FEAT):
                    v = in_v[f, ys, pl.ds(xc * 16, 16)]
                    plsc.store_scatter(dst, [lane, colf[f]], v)
                return carry3
            lax.fori_loop(0, 8, xc_body, 0)
            return carry2
        lax.fori_loop(0, 8, ys_body, 0)

        rb = (t * TEX_DIM + yt * 8) * TEX_DIM + xt * 128
        for ys in range(8):
            pltpu.sync_copy(
                out_v.at[pl.ds(ys * 128, 128)],
                table1.at[pl.ds(rb + ys * TEX_DIM, 128)])
        return carry

    lax.fori_loop(0, TPB, blk, 0)


def _sc_body(table, uvf, maskf, out, uv_v, tid_v, idx_v, w_v, rows_v, outT_v, sem):
    c = lax.axis_index("c")
    s = lax.axis_index("s")
    wid = s * 2 + c
    layer = wid // WPL
    base0 = (wid % WPL) * PPW
    lane = lax.broadcasted_iota(jnp.int32, (16,), 0)
    laneB = lane * B

    def chunk_body(ci, carry):
        base = base0 + ci * B
        pltpu.sync_copy(uvf.at[pl.ds(2 * layer, 2), pl.ds(base, B)], uv_v)
        pltpu.sync_copy(maskf.at[layer, pl.ds(base, B)], tid_v)

        def vec_body(r, carry2):
            for k in range(8):
                o = r * 128 + k * 16
                x = uv_v[0, pl.ds(o, 16)]
                y = uv_v[1, pl.ds(o, 16)]
                t = tid_v[pl.ds(o, 16)]
                ix = ((x + 1.0) * float(TEX_DIM) - 1.0) * 0.5
                iy = ((y + 1.0) * float(TEX_DIM) - 1.0) * 0.5
                ixf = ix.astype(jnp.int32).astype(jnp.float32)
                ix0 = jnp.where(ixf > ix, ixf - 1.0, ixf)
                iyf = iy.astype(jnp.int32).astype(jnp.float32)
                iy0 = jnp.where(iyf > iy, iyf - 1.0, iyf)
                wx1 = ix - ix0
                wy1 = iy - iy0
                wx0 = 1.0 - wx1
                wy0 = 1.0 - wy1
                valid = jnp.where(t >= 1, 1.0, 0.0)
                wy0 = wy0 * valid
                wy1 = wy1 * valid
                w_v[0, pl.ds(o, 16)] = wy0 * wx0
                w_v[1, pl.ds(o, 16)] = wy0 * wx1
                w_v[2, pl.ds(o, 16)] = wy1 * wx0
                w_v[3, pl.ds(o, 16)] = wy1 * wx1
                hi = float(TEX_DIM - 1)
                ix0c = jnp.clip(ix0, 0.0, hi).astype(jnp.int32)
                ix1c = jnp.clip(ix0 + 1.0, 0.0, hi).astype(jnp.int32)
                iy0c = jnp.clip(iy0, 0.0, hi).astype(jnp.int32)
                iy1c = jnp.clip(iy0 + 1.0, 0.0, hi).astype(jnp.int32)
                tb = t << 18
                r0 = tb + (iy0c << 9)
                r1 = tb + (iy1c << 9)
                col = pl.ds(k * 16, 16)
                idx_v[0 * RPT + r, col] = r0 + ix0c
                idx_v[1 * RPT + r, col] = r0 + ix1c
                idx_v[2 * RPT + r, col] = r1 + ix0c
                idx_v[3 * RPT + r, col] = r1 + ix1c
            return carry2

        lax.fori_loop(0, RPT, vec_body, 0)

        handles = []
        for j in range(NROW):
            handles.append(
                pltpu.async_copy(
                    table.at[idx_v.at[j]],
                    rows_v.at[pl.ds(j * 128, 128)],
                    sem,
                )
            )
        for h in handles:
            h.wait()

        def pix_body(g, carry2):
            o = g * 16
            w0v = w_v[0, pl.ds(o, 16)]
            w1v = w_v[1, pl.ds(o, 16)]
            w2v = w_v[2, pl.ds(o, 16)]
            w3v = w_v[3, pl.ds(o, 16)]
            for j in range(16):
                p = o + j
                acc = (rows_v[p, :] * w0v[j] + rows_v[B + p, :] * w1v[j]
                       + rows_v[2 * B + p, :] * w2v[j]
                       + rows_v[3 * B + p, :] * w3v[j])
                plsc.store_scatter(outT_v, [laneB + p], acc)
            return carry2

        lax.fori_loop(0, B // 16, pix_body, 0)

        for f in range(N_FEAT):
            pltpu.sync_copy(outT_v.at[pl.ds(f * B, B)],
                            out.at[layer, f, pl.ds(base, B)])
        return carry

    lax.fori_loop(0, NCHUNK, chunk_body, 0)


@functools.partial(jax.jit, static_argnames=())
def _run(data, uvf, maskf):
    mesh = plsc.VectorSubcoreMesh(
        core_axis_name="c", subcore_axis_name="s",
        num_cores=2, num_subcores=16)
    tr = pl.kernel(
        _tr_body,
        out_type=jax.ShapeDtypeStruct((N_TEX * TEX_DIM * TEX_DIM, N_FEAT),
                                      jnp.float32),
        mesh=mesh,
        scratch_types=[
            pltpu.VMEM((N_FEAT, 8, 128), jnp.float32),
            pltpu.VMEM((8 * 128, N_FEAT), jnp.float32),
            pltpu.SemaphoreType.DMA,
        ],
        compiler_params=pltpu.CompilerParams(
            needs_layout_passes=False, use_tc_tiling_on_sc=False),
    )
    table = tr(data)
    fn = pl.kernel(
        _sc_body,
        out_type=jax.ShapeDtypeStruct((N_LAYERS, N_FEAT, P), jnp.float32),
        mesh=mesh,
        scratch_types=[
            pltpu.VMEM((2, B), jnp.float32),
            pltpu.VMEM((B,), jnp.int32),
            pltpu.VMEM((NROW, 128), jnp.int32),
            pltpu.VMEM((4, B), jnp.float32),
            pltpu.VMEM((4 * B, N_FEAT), jnp.float32),
            pltpu.VMEM((N_FEAT * B,), jnp.float32),
            pltpu.SemaphoreType.DMA,
        ],
        compiler_params=pltpu.CompilerParams(
            needs_layout_passes=False, use_tc_tiling_on_sc=False),
    )
    return fn(table, uvf, maskf)


def kernel(data, uv_inputs, mask_inputs, world_positions, extrinsics):
    uvf = uv_inputs.reshape(2 * N_LAYERS, P)
    maskf = mask_inputs.reshape(N_LAYERS, P).astype(jnp.int32)
    dataT = data.reshape(N_TEX, N_FEAT, TEX_DIM // 8, 8,
                         TEX_DIM // 128, 128).swapaxes(3, 4)
    res = _run(dataT, uvf, maskf)
    return res.reshape(1, N_LAYERS * N_FEAT, H, W)


# speedup vs baseline: 1.3401x; 1.2020x over previous
import functools

import jax
import jax.numpy as jnp
from jax import lax
from jax.experimental import pallas as pl
from jax.experimental.pallas import tpu as pltpu
from jax.experimental.pallas import tpu_sc as plsc

N_TEX = 16
N_FEAT = 16
TEX_DIM = 512
N_LAYERS = 4
H = 512
W = 512
P = H * W
NW = 32
WPL = NW // N_LAYERS
PPW = P // WPL
B = 1024
NCHUNK = PPW // B
NROW = (4 * B) // 128
RPT = B // 128


TPB = 128


def _tr_body(data4, tableV, in_v, out_v, sem_i0, sem_i1, sem_o0, sem_o1):
    c = lax.axis_index("c")
    s = lax.axis_index("s")
    wid = s * 2 + c
    lane = lax.broadcasted_iota(jnp.int32, (16,), 0)
    colf = [jnp.full((16,), f, jnp.int32) for f in range(N_FEAT)]
    sem_i = (sem_i0, sem_i1)
    sem_o = (sem_o0, sem_o1)

    def gidx(bi):
        g = wid * TPB + bi
        t = g // 256
        rem = g % 256
        return t, rem // 4, rem % 4

    def in_desc(bi, b):
        t, yt, xt = gidx(bi)
        return pltpu.make_async_copy(
            data4.at[t, :, yt, xt], in_v.at[b], sem_i[b])

    def out_desc(bi, b):
        t, yt, xt = gidx(bi)
        return pltpu.make_async_copy(
            out_v.at[b], tableV.at[t, pl.ds(yt * 8, 8), xt], sem_o[b])

    def interleave(b):
        def ys_body(ys, carry2):
            def xc_body(xc, carry3):
                dst = out_v.at[b, ys, pl.ds(xc * 16, 16), :]
                for f in range(N_FEAT):
                    v = in_v[b, f, ys, pl.ds(xc * 16, 16)]
                    plsc.store_scatter(dst, [lane, colf[f]], v)
                return carry3
            lax.fori_loop(0, 8, xc_body, 0)
            return carry2
        lax.fori_loop(0, 8, ys_body, 0)

    in_desc(0, 0).start()
    in_desc(1, 1).start()
    in_desc(0, 0).wait()
    interleave(0)
    out_desc(0, 0).start()
    in_desc(2, 0).start()
    in_desc(1, 1).wait()
    interleave(1)
    out_desc(1, 1).start()
    in_desc(3, 1).start()

    def pair(i, carry):
        for b in range(2):
            bi = 2 * i + b
            in_desc(bi, b).wait()
            out_desc(bi - 2, b).wait()
            interleave(b)
            out_desc(bi, b).start()
            in_desc(bi + 2, b).start()
        return carry

    lax.fori_loop(1, TPB // 2 - 1, pair, 0)

    for bi in (TPB - 2, TPB - 1):
        b = bi % 2
        in_desc(bi, b).wait()
        out_desc(bi - 2, b).wait()
        interleave(b)
        out_desc(bi, b).start()
    out_desc(TPB - 2, 0).wait()
    out_desc(TPB - 1, 1).wait()


def _sc_body(table, uvf, maskf, out, uv_v, tid_v, idx_v, w_v, rows_v, outT_v, sem):
    c = lax.axis_index("c")
    s = lax.axis_index("s")
    wid = s * 2 + c
    layer = wid // WPL
    base0 = (wid % WPL) * PPW
    lane = lax.broadcasted_iota(jnp.int32, (16,), 0)
    laneB = lane * B

    def chunk_body(ci, carry):
        base = base0 + ci * B
        pltpu.sync_copy(uvf.at[pl.ds(2 * layer, 2), pl.ds(base, B)], uv_v)
        pltpu.sync_copy(maskf.at[layer, pl.ds(base, B)], tid_v)

        def vec_body(r, carry2):
            for k in range(8):
                o = r * 128 + k * 16
                x = uv_v[0, pl.ds(o, 16)]
                y = uv_v[1, pl.ds(o, 16)]
                t = tid_v[pl.ds(o, 16)]
                ix = ((x + 1.0) * float(TEX_DIM) - 1.0) * 0.5
                iy = ((y + 1.0) * float(TEX_DIM) - 1.0) * 0.5
                ixf = ix.astype(jnp.int32).astype(jnp.float32)
                ix0 = jnp.where(ixf > ix, ixf - 1.0, ixf)
                iyf = iy.astype(jnp.int32).astype(jnp.float32)
                iy0 = jnp.where(iyf > iy, iyf - 1.0, iyf)
                wx1 = ix - ix0
                wy1 = iy - iy0
                wx0 = 1.0 - wx1
                wy0 = 1.0 - wy1
                valid = jnp.where(t >= 1, 1.0, 0.0)
                wy0 = wy0 * valid
                wy1 = wy1 * valid
                w_v[0, pl.ds(o, 16)] = wy0 * wx0
                w_v[1, pl.ds(o, 16)] = wy0 * wx1
                w_v[2, pl.ds(o, 16)] = wy1 * wx0
                w_v[3, pl.ds(o, 16)] = wy1 * wx1
                hi = float(TEX_DIM - 1)
                ix0c = jnp.clip(ix0, 0.0, hi).astype(jnp.int32)
                ix1c = jnp.clip(ix0 + 1.0, 0.0, hi).astype(jnp.int32)
                iy0c = jnp.clip(iy0, 0.0, hi).astype(jnp.int32)
                iy1c = jnp.clip(iy0 + 1.0, 0.0, hi).astype(jnp.int32)
                tb = t << 18
                r0 = tb + (iy0c << 9)
                r1 = tb + (iy1c << 9)
                col = pl.ds(k * 16, 16)
                idx_v[0 * RPT + r, col] = r0 + ix0c
                idx_v[1 * RPT + r, col] = r0 + ix1c
                idx_v[2 * RPT + r, col] = r1 + ix0c
                idx_v[3 * RPT + r, col] = r1 + ix1c
            return carry2

        lax.fori_loop(0, RPT, vec_body, 0)

        handles = []
        for j in range(NROW):
            handles.append(
                pltpu.async_copy(
                    table.at[idx_v.at[j]],
                    rows_v.at[pl.ds(j * 128, 128)],
                    sem,
                )
            )
        for h in handles:
            h.wait()

        def pix_body(g, carry2):
            o = g * 16
            w0v = w_v[0, pl.ds(o, 16)]
            w1v = w_v[1, pl.ds(o, 16)]
            w2v = w_v[2, pl.ds(o, 16)]
            w3v = w_v[3, pl.ds(o, 16)]
            for j in range(16):
                p = o + j
                acc = (rows_v[p, :] * w0v[j] + rows_v[B + p, :] * w1v[j]
                       + rows_v[2 * B + p, :] * w2v[j]
                       + rows_v[3 * B + p, :] * w3v[j])
                plsc.store_scatter(outT_v, [laneB + p], acc)
            return carry2

        lax.fori_loop(0, B // 16, pix_body, 0)

        for f in range(N_FEAT):
            pltpu.sync_copy(outT_v.at[pl.ds(f * B, B)],
                            out.at[layer, f, pl.ds(base, B)])
        return carry

    lax.fori_loop(0, NCHUNK, chunk_body, 0)


@functools.partial(jax.jit, static_argnames=())
def _run(data, uvf, maskf):
    mesh = plsc.VectorSubcoreMesh(
        core_axis_name="c", subcore_axis_name="s",
        num_cores=2, num_subcores=16)
    tr = pl.kernel(
        _tr_body,
        out_type=jax.ShapeDtypeStruct(
            (N_TEX, TEX_DIM, TEX_DIM // 128, 128, N_FEAT), jnp.float32),
        mesh=mesh,
        scratch_types=[
            pltpu.VMEM((2, N_FEAT, 8, 128), jnp.float32),
            pltpu.VMEM((2, 8, 128, N_FEAT), jnp.float32),
            pltpu.SemaphoreType.DMA,
            pltpu.SemaphoreType.DMA,
            pltpu.SemaphoreType.DMA,
            pltpu.SemaphoreType.DMA,
        ],
        compiler_params=pltpu.CompilerParams(
            needs_layout_passes=False, use_tc_tiling_on_sc=False),
    )
    table = tr(data).reshape(N_TEX * TEX_DIM * TEX_DIM, N_FEAT)
    fn = pl.kernel(
        _sc_body,
        out_type=jax.ShapeDtypeStruct((N_LAYERS, N_FEAT, P), jnp.float32),
        mesh=mesh,
        scratch_types=[
            pltpu.VMEM((2, B), jnp.float32),
            pltpu.VMEM((B,), jnp.int32),
            pltpu.VMEM((NROW, 128), jnp.int32),
            pltpu.VMEM((4, B), jnp.float32),
            pltpu.VMEM((4 * B, N_FEAT), jnp.float32),
            pltpu.VMEM((N_FEAT * B,), jnp.float32),
            pltpu.SemaphoreType.DMA,
        ],
        compiler_params=pltpu.CompilerParams(
            needs_layout_passes=False, use_tc_tiling_on_sc=False),
    )
    return fn(table, uvf, maskf)


def kernel(data, uv_inputs, mask_inputs, world_positions, extrinsics):
    uvf = uv_inputs.reshape(2 * N_LAYERS, P)
    maskf = mask_inputs.reshape(N_LAYERS, P).astype(jnp.int32)
    dataT = data.reshape(N_TEX, N_FEAT, TEX_DIM // 8, 8,
                         TEX_DIM // 128, 128).swapaxes(3, 4)
    res = _run(dataT, uvf, maskf)
    return res.reshape(1, N_LAYERS * N_FEAT, H, W)


# speedup vs baseline: 1.4159x; 1.0565x over previous
import functools

import jax
import jax.numpy as jnp
from jax import lax
from jax.experimental import pallas as pl
from jax.experimental.pallas import tpu as pltpu
from jax.experimental.pallas import tpu_sc as plsc

N_TEX = 16
N_FEAT = 16
TEX_DIM = 512
N_LAYERS = 4
H = 512
W = 512
P = H * W
NW = 32
WPL = NW // N_LAYERS
PPW = P // WPL
B = 1024
NCHUNK = PPW // B
NROW = (4 * B) // 128
RPT = B // 128


TPB = 128


def _tr_body(data4, tableV, in_v, out_v, sem_i0, sem_i1, sem_o0, sem_o1):
    c = lax.axis_index("c")
    s = lax.axis_index("s")
    wid = s * 2 + c
    lane = lax.broadcasted_iota(jnp.int32, (16,), 0)
    colf = [jnp.full((16,), f, jnp.int32) for f in range(N_FEAT)]
    sem_i = (sem_i0, sem_i1)
    sem_o = (sem_o0, sem_o1)

    def gidx(bi):
        g = wid * TPB + bi
        t = g // 256
        rem = g % 256
        return t, rem // 4, rem % 4

    def in_desc(bi, b):
        t, yt, xt = gidx(bi)
        return pltpu.make_async_copy(
            data4.at[t, :, yt, xt], in_v.at[b], sem_i[b])

    def out_desc(bi, b):
        t, yt, xt = gidx(bi)
        return pltpu.make_async_copy(
            out_v.at[b], tableV.at[t, pl.ds(yt * 8, 8), xt], sem_o[b])

    def interleave(b):
        def ys_body(ys, carry2):
            def xc_body(xc, carry3):
                dst = out_v.at[b, ys, pl.ds(xc * 16, 16), :]
                for f in range(N_FEAT):
                    v = in_v[b, f, ys, pl.ds(xc * 16, 16)]
                    plsc.store_scatter(dst, [lane, colf[f]], v)
                return carry3
            lax.fori_loop(0, 8, xc_body, 0)
            return carry2
        lax.fori_loop(0, 8, ys_body, 0)

    in_desc(0, 0).start()
    in_desc(1, 1).start()
    in_desc(0, 0).wait()
    interleave(0)
    out_desc(0, 0).start()
    in_desc(2, 0).start()
    in_desc(1, 1).wait()
    interleave(1)
    out_desc(1, 1).start()
    in_desc(3, 1).start()

    def pair(i, carry):
        for b in range(2):
            bi = 2 * i + b
            in_desc(bi, b).wait()
            out_desc(bi - 2, b).wait()
            interleave(b)
            out_desc(bi, b).start()
            in_desc(bi + 2, b).start()
        return carry

    lax.fori_loop(1, TPB // 2 - 1, pair, 0)

    for bi in (TPB - 2, TPB - 1):
        b = bi % 2
        in_desc(bi, b).wait()
        out_desc(bi - 2, b).wait()
        interleave(b)
        out_desc(bi, b).start()
    out_desc(TPB - 2, 0).wait()
    out_desc(TPB - 1, 1).wait()


def _sc_body(table, uvf, maskf, out, uv_v, tid_v, idx_v, w_v, rows_v, outT_v,
             sem_g, sem_u0, sem_u1, sem_t0, sem_t1):
    c = lax.axis_index("c")
    s = lax.axis_index("s")
    wid = s * 2 + c
    layer = wid // WPL
    base0 = (wid % WPL) * PPW
    lane = lax.broadcasted_iota(jnp.int32, (16,), 0)
    sem_u = (sem_u0, sem_u1)
    sem_t = (sem_t0, sem_t1)

    def uv_descs(ci, b):
        base = base0 + ci * B
        return (
            pltpu.make_async_copy(
                uvf.at[pl.ds(2 * layer, 2), pl.ds(base, B)],
                uv_v.at[b], sem_u[b]),
            pltpu.make_async_copy(
                maskf.at[layer, pl.ds(base, B)], tid_v.at[b], sem_u[b]),
        )

    def out_desc(ci, b):
        base = base0 + ci * B
        return pltpu.make_async_copy(
            outT_v.at[b], out.at[layer, :, pl.ds(base, B)], sem_t[b])

    def compute_idx(b):
        def vec_body(r, carry2):
            for k in range(8):
                o = r * 128 + k * 16
                x = uv_v[b, 0, pl.ds(o, 16)]
                y = uv_v[b, 1, pl.ds(o, 16)]
                t = tid_v[b, pl.ds(o, 16)]
                ix = ((x + 1.0) * float(TEX_DIM) - 1.0) * 0.5
                iy = ((y + 1.0) * float(TEX_DIM) - 1.0) * 0.5
                ixf = ix.astype(jnp.int32).astype(jnp.float32)
                ix0 = jnp.where(ixf > ix, ixf - 1.0, ixf)
                iyf = iy.astype(jnp.int32).astype(jnp.float32)
                iy0 = jnp.where(iyf > iy, iyf - 1.0, iyf)
                wx1 = ix - ix0
                wy1 = iy - iy0
                wx0 = 1.0 - wx1
                wy0 = 1.0 - wy1
                valid = jnp.where(t >= 1, 1.0, 0.0)
                wy0 = wy0 * valid
                wy1 = wy1 * valid
                w_v[0, pl.ds(o, 16)] = wy0 * wx0
                w_v[1, pl.ds(o, 16)] = wy0 * wx1
                w_v[2, pl.ds(o, 16)] = wy1 * wx0
                w_v[3, pl.ds(o, 16)] = wy1 * wx1
                hi = float(TEX_DIM - 1)
                ix0c = jnp.clip(ix0, 0.0, hi).astype(jnp.int32)
                ix1c = jnp.clip(ix0 + 1.0, 0.0, hi).astype(jnp.int32)
                iy0c = jnp.clip(iy0, 0.0, hi).astype(jnp.int32)
                iy1c = jnp.clip(iy0 + 1.0, 0.0, hi).astype(jnp.int32)
                tb = t << 18
                r0 = tb + (iy0c << 9)
                r1 = tb + (iy1c << 9)
                col = pl.ds(k * 16, 16)
                idx_v[0 * RPT + r, col] = r0 + ix0c
                idx_v[1 * RPT + r, col] = r0 + ix1c
                idx_v[2 * RPT + r, col] = r1 + ix0c
                idx_v[3 * RPT + r, col] = r1 + ix1c
            return carry2

        lax.fori_loop(0, RPT, vec_body, 0)

    def gather_and_blend(b):
        handles = []
        for j in range(NROW):
            handles.append(
                pltpu.async_copy(
                    table.at[idx_v.at[j]],
                    rows_v.at[pl.ds(j * 128, 128)],
                    sem_g,
                )
            )
        for h in handles:
            h.wait()

        def pix_body(g, carry2):
            o = g * 16
            w0v = w_v[0, pl.ds(o, 16)]
            w1v = w_v[1, pl.ds(o, 16)]
            w2v = w_v[2, pl.ds(o, 16)]
            w3v = w_v[3, pl.ds(o, 16)]
            pvec = jnp.full((16,), o, jnp.int32)
            for j in range(16):
                p = o + j
                acc = (rows_v[p, :] * w0v[j] + rows_v[B + p, :] * w1v[j]
                       + rows_v[2 * B + p, :] * w2v[j]
                       + rows_v[3 * B + p, :] * w3v[j])
                plsc.store_scatter(outT_v.at[b], [lane, pvec + j], acc)
            return carry2

        lax.fori_loop(0, B // 16, pix_body, 0)

    for d in uv_descs(0, 0) + uv_descs(1, 1):
        d.start()
    for ci in (0, 1):
        b = ci
        for d in uv_descs(ci, b):
            d.wait()
        compute_idx(b)
        uv_descs(ci + 2, b)[0].start()
        uv_descs(ci + 2, b)[1].start()
        gather_and_blend(b)
        out_desc(ci, b).start()

    def pair(i, carry):
        for b in range(2):
            ci = 2 * i + b
            for d in uv_descs(ci, b):
                d.wait()
            compute_idx(b)
            uv_descs(ci + 2, b)[0].start()
            uv_descs(ci + 2, b)[1].start()
            out_desc(ci - 2, b).wait()
            gather_and_blend(b)
            out_desc(ci, b).start()
        return carry

    lax.fori_loop(1, NCHUNK // 2 - 1, pair, 0)

    for ci in (NCHUNK - 2, NCHUNK - 1):
        b = ci % 2
        for d in uv_descs(ci, b):
            d.wait()
        compute_idx(b)
        out_desc(ci - 2, b).wait()
        gather_and_blend(b)
        out_desc(ci, b).start()
    out_desc(NCHUNK - 2, 0).wait()
    out_desc(NCHUNK - 1, 1).wait()


@functools.partial(jax.jit, static_argnames=())
def _run(data, uvf, maskf):
    mesh = plsc.VectorSubcoreMesh(
        core_axis_name="c", subcore_axis_name="s",
        num_cores=2, num_subcores=16)
    tr = pl.kernel(
        _tr_body,
        out_type=jax.ShapeDtypeStruct(
            (N_TEX, TEX_DIM, TEX_DIM // 128, 128, N_FEAT), jnp.float32),
        mesh=mesh,
        scratch_types=[
            pltpu.VMEM((2, N_FEAT, 8, 128), jnp.float32),
            pltpu.VMEM((2, 8, 128, N_FEAT), jnp.float32),
            pltpu.SemaphoreType.DMA,
            pltpu.SemaphoreType.DMA,
            pltpu.SemaphoreType.DMA,
            pltpu.SemaphoreType.DMA,
        ],
        compiler_params=pltpu.CompilerParams(
            needs_layout_passes=False, use_tc_tiling_on_sc=False),
    )
    table = tr(data).reshape(N_TEX * TEX_DIM * TEX_DIM, N_FEAT)
    fn = pl.kernel(
        _sc_body,
        out_type=jax.ShapeDtypeStruct((N_LAYERS, N_FEAT, P), jnp.float32),
        mesh=mesh,
        scratch_types=[
            pltpu.VMEM((2, 2, B), jnp.float32),
            pltpu.VMEM((2, B), jnp.int32),
            pltpu.VMEM((NROW, 128), jnp.int32),
            pltpu.VMEM((4, B), jnp.float32),
            pltpu.VMEM((4 * B, N_FEAT), jnp.float32),
            pltpu.VMEM((2, N_FEAT, B), jnp.float32),
            pltpu.SemaphoreType.DMA,
            pltpu.SemaphoreType.DMA,
            pltpu.SemaphoreType.DMA,
            pltpu.SemaphoreType.DMA,
            pltpu.SemaphoreType.DMA,
        ],
        compiler_params=pltpu.CompilerParams(
            needs_layout_passes=False, use_tc_tiling_on_sc=False),
    )
    return fn(table, uvf, maskf)


def kernel(data, uv_inputs, mask_inputs, world_positions, extrinsics):
    uvf = uv_inputs.reshape(2 * N_LAYERS, P)
    maskf = mask_inputs.reshape(N_LAYERS, P).astype(jnp.int32)
    dataT = data.reshape(N_TEX, N_FEAT, TEX_DIM // 8, 8,
                         TEX_DIM // 128, 128).swapaxes(3, 4)
    res = _run(dataT, uvf, maskf)
    return res.reshape(1, N_LAYERS * N_FEAT, H, W)


# speedup vs baseline: 1.4965x; 1.0569x over previous
import functools

import jax
import jax.numpy as jnp
from jax import lax
from jax.experimental import pallas as pl
from jax.experimental.pallas import tpu as pltpu
from jax.experimental.pallas import tpu_sc as plsc

N_TEX = 16
N_FEAT = 16
TEX_DIM = 512
N_LAYERS = 4
H = 512
W = 512
P = H * W
NW = 32
WPL = NW // N_LAYERS
PPW = P // WPL
B = 512
NCHUNK = PPW // B
NROW = (4 * B) // 128
RPT = B // 128


TPB = 128


def _tr_body(data4, tableV, in_v, out_v, sem_i0, sem_i1, sem_o0, sem_o1):
    c = lax.axis_index("c")
    s = lax.axis_index("s")
    wid = s * 2 + c
    lane = lax.broadcasted_iota(jnp.int32, (16,), 0)
    colf = [jnp.full((16,), f, jnp.int32) for f in range(N_FEAT)]
    sem_i = (sem_i0, sem_i1)
    sem_o = (sem_o0, sem_o1)

    def gidx(bi):
        g = wid * TPB + bi
        t = g // 256
        rem = g % 256
        return t, rem // 4, rem % 4

    def in_desc(bi, b):
        t, yt, xt = gidx(bi)
        return pltpu.make_async_copy(
            data4.at[t, :, yt, xt], in_v.at[b], sem_i[b])

    def out_desc(bi, b):
        t, yt, xt = gidx(bi)
        return pltpu.make_async_copy(
            out_v.at[b], tableV.at[t, pl.ds(yt * 8, 8), xt], sem_o[b])

    def interleave(b):
        def ys_body(ys, carry2):
            def xc_body(xc, carry3):
                dst = out_v.at[b, ys, pl.ds(xc * 16, 16), :]
                for f in range(N_FEAT):
                    v = in_v[b, f, ys, pl.ds(xc * 16, 16)]
                    plsc.store_scatter(dst, [lane, colf[f]], v)
                return carry3
            lax.fori_loop(0, 8, xc_body, 0)
            return carry2
        lax.fori_loop(0, 8, ys_body, 0)

    in_desc(0, 0).start()
    in_desc(1, 1).start()
    in_desc(0, 0).wait()
    interleave(0)
    out_desc(0, 0).start()
    in_desc(2, 0).start()
    in_desc(1, 1).wait()
    interleave(1)
    out_desc(1, 1).start()
    in_desc(3, 1).start()

    def pair(i, carry):
        for b in range(2):
            bi = 2 * i + b
            in_desc(bi, b).wait()
            out_desc(bi - 2, b).wait()
            interleave(b)
            out_desc(bi, b).start()
            in_desc(bi + 2, b).start()
        return carry

    lax.fori_loop(1, TPB // 2 - 1, pair, 0)

    for bi in (TPB - 2, TPB - 1):
        b = bi % 2
        in_desc(bi, b).wait()
        out_desc(bi - 2, b).wait()
        interleave(b)
        out_desc(bi, b).start()
    out_desc(TPB - 2, 0).wait()
    out_desc(TPB - 1, 1).wait()


def _sc_body(table, uvf, maskf, out, uv_v, tid_v, idx0_v, idx1_v, w_v,
             rows0_v, rows1_v, outT_v,
             sem_g0, sem_g1, sem_u0, sem_u1, sem_t0, sem_t1):
    c = lax.axis_index("c")
    s = lax.axis_index("s")
    wid = s * 2 + c
    layer = wid // WPL
    base0 = (wid % WPL) * PPW
    lane = lax.broadcasted_iota(jnp.int32, (16,), 0)
    idx_b = (idx0_v, idx1_v)
    rows_b = (rows0_v, rows1_v)
    sem_g = (sem_g0, sem_g1)
    sem_u = (sem_u0, sem_u1)
    sem_t = (sem_t0, sem_t1)

    def uv_descs(ci, b):
        base = base0 + ci * B
        return (
            pltpu.make_async_copy(
                uvf.at[pl.ds(2 * layer, 2), pl.ds(base, B)],
                uv_v.at[b], sem_u[b]),
            pltpu.make_async_copy(
                maskf.at[layer, pl.ds(base, B)], tid_v.at[b], sem_u[b]),
        )

    def out_desc(ci, b):
        base = base0 + ci * B
        return pltpu.make_async_copy(
            outT_v.at[b], out.at[layer, :, pl.ds(base, B)], sem_t[b])

    def compute_idx(b):
        def vec_body(r, carry2):
            for k in range(8):
                o = r * 128 + k * 16
                x = uv_v[b, 0, pl.ds(o, 16)]
                y = uv_v[b, 1, pl.ds(o, 16)]
                t = tid_v[b, pl.ds(o, 16)]
                ix = ((x + 1.0) * float(TEX_DIM) - 1.0) * 0.5
                iy = ((y + 1.0) * float(TEX_DIM) - 1.0) * 0.5
                ixf = ix.astype(jnp.int32).astype(jnp.float32)
                ix0 = jnp.where(ixf > ix, ixf - 1.0, ixf)
                iyf = iy.astype(jnp.int32).astype(jnp.float32)
                iy0 = jnp.where(iyf > iy, iyf - 1.0, iyf)
                wx1 = ix - ix0
                wy1 = iy - iy0
                wx0 = 1.0 - wx1
                wy0 = 1.0 - wy1
                valid = jnp.where(t >= 1, 1.0, 0.0)
                wy0 = wy0 * valid
                wy1 = wy1 * valid
                w_v[b, 0, pl.ds(o, 16)] = wy0 * wx0
                w_v[b, 1, pl.ds(o, 16)] = wy0 * wx1
                w_v[b, 2, pl.ds(o, 16)] = wy1 * wx0
                w_v[b, 3, pl.ds(o, 16)] = wy1 * wx1
                hi = float(TEX_DIM - 1)
                ix0c = jnp.clip(ix0, 0.0, hi).astype(jnp.int32)
                ix1c = jnp.clip(ix0 + 1.0, 0.0, hi).astype(jnp.int32)
                iy0c = jnp.clip(iy0, 0.0, hi).astype(jnp.int32)
                iy1c = jnp.clip(iy0 + 1.0, 0.0, hi).astype(jnp.int32)
                tb = t << 18
                r0 = tb + (iy0c << 9)
                r1 = tb + (iy1c << 9)
                col = pl.ds(k * 16, 16)
                idx_b[b][0 * RPT + r, col] = r0 + ix0c
                idx_b[b][1 * RPT + r, col] = r0 + ix1c
                idx_b[b][2 * RPT + r, col] = r1 + ix0c
                idx_b[b][3 * RPT + r, col] = r1 + ix1c
            return carry2

        lax.fori_loop(0, RPT, vec_body, 0)

    def fire_gathers(b):
        return [
            pltpu.async_copy(
                table.at[idx_b[b].at[j]],
                rows_b[b].at[pl.ds(j * 128, 128)],
                sem_g[b],
            )
            for j in range(NROW)
        ]

    def blend(b):
        def pix_body(g, carry2):
            o = g * 16
            w0v = w_v[b, 0, pl.ds(o, 16)]
            w1v = w_v[b, 1, pl.ds(o, 16)]
            w2v = w_v[b, 2, pl.ds(o, 16)]
            w3v = w_v[b, 3, pl.ds(o, 16)]
            pvec = jnp.full((16,), o, jnp.int32)
            for j in range(16):
                p = o + j
                acc = (rows_b[b][p, :] * w0v[j]
                       + rows_b[b][B + p, :] * w1v[j]
                       + rows_b[b][2 * B + p, :] * w2v[j]
                       + rows_b[b][3 * B + p, :] * w3v[j])
                plsc.store_scatter(outT_v.at[b], [lane, pvec + j], acc)
            return carry2

        lax.fori_loop(0, B // 16, pix_body, 0)

    def front(ci, b, prefetch):
        for d in uv_descs(ci, b):
            d.wait()
        compute_idx(b)
        if prefetch:
            uv_descs(ci + 2, b)[0].start()
            uv_descs(ci + 2, b)[1].start()
        return fire_gathers(b)

    def back(ci, b, handles, out_wait):
        for h in handles:
            h.wait()
        if out_wait:
            out_desc(ci - 2, b).wait()
        blend(b)
        out_desc(ci, b).start()

    def do_pair(c0, out_wait, prefetch):
        h0 = front(c0, 0, prefetch)
        h1 = front(c0 + 1, 1, prefetch)
        back(c0, 0, h0, out_wait)
        back(c0 + 1, 1, h1, out_wait)

    for d in uv_descs(0, 0) + uv_descs(1, 1):
        d.start()
    do_pair(0, False, True)

    def pair(i, carry):
        do_pair(2 * i, True, True)
        return carry

    lax.fori_loop(1, NCHUNK // 2 - 1, pair, 0)

    do_pair(NCHUNK - 2, True, False)
    out_desc(NCHUNK - 2, 0).wait()
    out_desc(NCHUNK - 1, 1).wait()


@functools.partial(jax.jit, static_argnames=())
def _run(data, uvf, maskf):
    mesh = plsc.VectorSubcoreMesh(
        core_axis_name="c", subcore_axis_name="s",
        num_cores=2, num_subcores=16)
    tr = pl.kernel(
        _tr_body,
        out_type=jax.ShapeDtypeStruct(
            (N_TEX, TEX_DIM, TEX_DIM // 128, 128, N_FEAT), jnp.float32),
        mesh=mesh,
        scratch_types=[
            pltpu.VMEM((2, N_FEAT, 8, 128), jnp.float32),
            pltpu.VMEM((2, 8, 128, N_FEAT), jnp.float32),
            pltpu.SemaphoreType.DMA,
            pltpu.SemaphoreType.DMA,
            pltpu.SemaphoreType.DMA,
            pltpu.SemaphoreType.DMA,
        ],
        compiler_params=pltpu.CompilerParams(
            needs_layout_passes=False, use_tc_tiling_on_sc=False),
    )
    table = tr(data).reshape(N_TEX * TEX_DIM * TEX_DIM, N_FEAT)
    fn = pl.kernel(
        _sc_body,
        out_type=jax.ShapeDtypeStruct((N_LAYERS, N_FEAT, P), jnp.float32),
        mesh=mesh,
        scratch_types=[
            pltpu.VMEM((2, 2, B), jnp.float32),
            pltpu.VMEM((2, B), jnp.int32),
            pltpu.VMEM((NROW, 128), jnp.int32),
            pltpu.VMEM((NROW, 128), jnp.int32),
            pltpu.VMEM((2, 4, B), jnp.float32),
            pltpu.VMEM((4 * B, N_FEAT), jnp.float32),
            pltpu.VMEM((4 * B, N_FEAT), jnp.float32),
            pltpu.VMEM((2, N_FEAT, B), jnp.float32),
            pltpu.SemaphoreType.DMA,
            pltpu.SemaphoreType.DMA,
            pltpu.SemaphoreType.DMA,
            pltpu.SemaphoreType.DMA,
            pltpu.SemaphoreType.DMA,
            pltpu.SemaphoreType.DMA,
        ],
        compiler_params=pltpu.CompilerParams(
            needs_layout_passes=False, use_tc_tiling_on_sc=False),
    )
    return fn(table, uvf, maskf)


def kernel(data, uv_inputs, mask_inputs, world_positions, extrinsics):
    uvf = uv_inputs.reshape(2 * N_LAYERS, P)
    maskf = mask_inputs.reshape(N_LAYERS, P).astype(jnp.int32)
    dataT = data.reshape(N_TEX, N_FEAT, TEX_DIM // 8, 8,
                         TEX_DIM // 128, 128).swapaxes(3, 4)
    res = _run(dataT, uvf, maskf)
    return res.reshape(1, N_LAYERS * N_FEAT, H, W)
